# baseline (speedup 1.0000x reference)
import functools

import jax
import jax.numpy as jnp
from jax import lax
from jax.experimental import pallas as pl
from jax.experimental.pallas import tpu as pltpu
from jax.experimental.pallas import tpu_sc as plsc

_LANES = 16
_ROW_UNROLL = 1
_NSUB = 2


@functools.lru_cache(maxsize=None)
def _build(bs: int, seq_len: int, dim: int):
    info = plsc.get_sparse_core_info()
    nc, ns = info.num_cores, info.num_subcores
    nw = nc * ns
    num_rows = bs * seq_len
    assert num_rows % (nw * _NSUB) == 0
    chunk = num_rows // nw
    sub = chunk // _NSUB
    assert sub % 8 == 0 and sub % _ROW_UNROLL == 0
    assert seq_len % chunk == 0 and dim % _LANES == 0
    tiles_per_b = seq_len // chunk
    nslots = seq_len // (chunk * nc)
    assert nslots >= 1 and ns % nslots == 0

    mesh = plsc.VectorSubcoreMesh(core_axis_name="c", subcore_axis_name="s")

    @functools.partial(
        pl.kernel,
        mesh=mesh,
        out_type=jax.ShapeDtypeStruct((bs, seq_len, dim), jnp.float32),
        scratch_types=[
            pltpu.VMEM((chunk,), jnp.int32),
            pltpu.VMEM((chunk, dim), jnp.float32),
            pltpu.VMEM((chunk, dim), jnp.float32),
            pltpu.VMEM_SHARED((nslots, chunk, dim), jnp.float32),
            pltpu.SemaphoreType.DMA,
            pltpu.SemaphoreType.DMA,
            pltpu.SemaphoreType.DMA,
        ],
    )
    def embed(idx_hbm, tok_hbm, pos_hbm, out_hbm, idx_v, rows_v, pos_v,
              pos_sh, gsem, osem, ssem):
        cid = lax.axis_index("c")
        sid = lax.axis_index("s")
        wid = sid * nc + cid
        b = wid // tiles_per_b
        col = (wid % tiles_per_b) * chunk

        @pl.when(sid < nslots)
        def _stage():
            start = (nc * sid + cid) * chunk
            pltpu.async_copy(
                pos_hbm.at[pl.ds(start, chunk)], pos_sh.at[sid], ssem).wait()

        pltpu.sync_copy(idx_hbm.at[b, pl.ds(col, chunk)], idx_v)
        gathers = []
        for k in range(_NSUB):
            gathers.append(pltpu.async_copy(
                tok_hbm.at[idx_v.at[pl.ds(k * sub, sub)]],
                rows_v.at[pl.ds(k * sub, sub)], gsem))
        plsc.subcore_barrier()
        pltpu.sync_copy(pos_sh.at[sid % nslots], pos_v)

        stores = []
        for k in range(_NSUB):
            gathers[k].wait()

            def add_rows(t, _, k=k):
                i = k * sub + t // (dim // _LANES)
                sl = pl.ds((t % (dim // _LANES)) * _LANES, _LANES)
                plsc.addupdate(rows_v.at[i, sl], pos_v[i, sl])
                return 0

            lax.fori_loop(0, sub * (dim // _LANES), add_rows, 0)
            stores.append(pltpu.async_copy(
                rows_v.at[pl.ds(k * sub, sub)],
                out_hbm.at[b, pl.ds(col + k * sub, sub)], osem))
        for st in stores:
            st.wait()

    return embed


def kernel(tok_idx, tok_table, pos_table):
    bs, seq_len = tok_idx.shape
    dim = tok_table.shape[1]
    embed = _build(bs, seq_len, dim)
    return embed(tok_idx.astype(jnp.int32), tok_table, pos_table)


# speedup vs baseline: 1.2257x; 1.2257x over previous
import functools

import jax
import jax.numpy as jnp
from jax import lax
from jax.experimental import pallas as pl
from jax.experimental.pallas import tpu as pltpu
from jax.experimental.pallas import tpu_sc as plsc

_LANES = 16
_ROW_UNROLL = 1
_NSUB = 2


@functools.lru_cache(maxsize=None)
def _build(bs: int, seq_len: int, dim: int):
    info = plsc.get_sparse_core_info()
    nc, ns = info.num_cores, info.num_subcores
    nw = nc * ns
    num_rows = bs * seq_len
    assert num_rows % (nw * _NSUB) == 0
    chunk = num_rows // nw
    sub = chunk // _NSUB
    assert sub % 8 == 0 and sub % _ROW_UNROLL == 0
    assert seq_len % chunk == 0 and dim % _LANES == 0
    tiles_per_b = seq_len // chunk
    nslots = seq_len // (chunk * nc)
    assert nslots >= 1 and ns % nslots == 0

    mesh = plsc.VectorSubcoreMesh(core_axis_name="c", subcore_axis_name="s")

    @functools.partial(
        pl.kernel,
        mesh=mesh,
        out_type=jax.ShapeDtypeStruct((bs, seq_len, dim), jnp.float32),
        scratch_types=[
            pltpu.VMEM((chunk,), jnp.int32),
            pltpu.VMEM((chunk, dim), jnp.float32),
            pltpu.VMEM((chunk, dim), jnp.float32),
            pltpu.VMEM_SHARED((nslots, chunk, dim), jnp.float32),
            pltpu.SemaphoreType.DMA,
            pltpu.SemaphoreType.DMA,
            pltpu.SemaphoreType.DMA,
        ],
    )
    def embed(idx_hbm, tok_hbm, pos_hbm, out_hbm, idx_v, rows_v, pos_v,
              pos_sh, gsem, osem, ssem):
        cid = lax.axis_index("c")
        sid = lax.axis_index("s")
        wid = sid * nc + cid
        b = wid // tiles_per_b
        col = (wid % tiles_per_b) * chunk

        @pl.when(sid < nslots)
        def _stage():
            start = (nc * sid + cid) * chunk
            pltpu.async_copy(
                pos_hbm.at[pl.ds(start, chunk)], pos_sh.at[sid], ssem).wait()

        pltpu.sync_copy(idx_hbm.at[b, pl.ds(col, chunk)], idx_v)
        gathers = []
        for k in range(_NSUB):
            gathers.append(pltpu.async_copy(
                tok_hbm.at[idx_v.at[pl.ds(k * sub, sub)]],
                rows_v.at[pl.ds(k * sub, sub)], gsem))
        plsc.subcore_barrier()
        pltpu.sync_copy(pos_sh.at[sid % nslots], pos_v)

        stores = []
        for k in range(_NSUB):
            gathers[k].wait()

            def add_rows(i, _, k=k):
                r = k * sub + i
                for j in range(dim // _LANES):
                    sl = pl.ds(j * _LANES, _LANES)
                    plsc.addupdate(rows_v.at[r, sl], pos_v[r, sl])
                return 0

            lax.fori_loop(0, sub, add_rows, 0)
            stores.append(pltpu.async_copy(
                rows_v.at[pl.ds(k * sub, sub)],
                out_hbm.at[b, pl.ds(col + k * sub, sub)], osem))
        for st in stores:
            st.wait()

    return embed


def kernel(tok_idx, tok_table, pos_table):
    bs, seq_len = tok_idx.shape
    dim = tok_table.shape[1]
    embed = _build(bs, seq_len, dim)
    return embed(tok_idx.astype(jnp.int32), tok_table, pos_table)


# speedup vs baseline: 1.2566x; 1.0253x over previous
import functools

import jax
import jax.numpy as jnp
from jax import lax
from jax.experimental import pallas as pl
from jax.experimental.pallas import tpu as pltpu
from jax.experimental.pallas import tpu_sc as plsc

_LANES = 16
_ROW_UNROLL = 1
_NSUB = 2


@functools.lru_cache(maxsize=None)
def _build(bs: int, seq_len: int, dim: int):
    info = plsc.get_sparse_core_info()
    nc, ns = info.num_cores, info.num_subcores
    nw = nc * ns
    num_rows = bs * seq_len
    assert num_rows % (nw * _NSUB) == 0
    chunk = num_rows // nw
    sub = chunk // _NSUB
    assert sub % 8 == 0 and sub % _ROW_UNROLL == 0
    assert seq_len % chunk == 0 and dim % _LANES == 0
    tiles_per_b = seq_len // chunk
    nslots = seq_len // (chunk * nc)
    assert nslots >= 1 and ns % nslots == 0

    mesh = plsc.VectorSubcoreMesh(core_axis_name="c", subcore_axis_name="s")

    @functools.partial(
        pl.kernel,
        mesh=mesh,
        out_type=jax.ShapeDtypeStruct((bs, seq_len, dim), jnp.float32),
        scratch_types=[
            pltpu.VMEM((chunk,), jnp.int32),
            pltpu.VMEM((chunk, dim), jnp.float32),
            pltpu.VMEM((chunk, dim), jnp.float32),
            pltpu.VMEM_SHARED((nslots, chunk, dim), jnp.float32),
            pltpu.SemaphoreType.DMA,
            pltpu.SemaphoreType.DMA,
            pltpu.SemaphoreType.DMA,
            pltpu.SemaphoreType.DMA,
        ],
    )
    def embed(idx_hbm, tok_hbm, pos_hbm, out_hbm, idx_v, rows_v, pos_v,
              pos_sh, gsem, osem, ssem, psem):
        cid = lax.axis_index("c")
        sid = lax.axis_index("s")
        wid = sid * nc + cid
        b = wid // tiles_per_b
        col = (wid % tiles_per_b) * chunk

        pltpu.sync_copy(idx_hbm.at[b, pl.ds(col, chunk)], idx_v)
        gathers = []
        for k in range(_NSUB):
            gathers.append(pltpu.async_copy(
                tok_hbm.at[idx_v.at[pl.ds(k * sub, sub)]],
                rows_v.at[pl.ds(k * sub, sub)], gsem))

        @pl.when(sid < nslots)
        def _stage():
            start = (nc * sid + cid) * chunk
            pltpu.async_copy(
                pos_hbm.at[pl.ds(start, chunk)], pos_sh.at[sid], ssem).wait()

        plsc.subcore_barrier()
        poscps = []
        for k in range(_NSUB):
            poscps.append(pltpu.async_copy(
                pos_sh.at[sid % nslots, pl.ds(k * sub, sub)],
                pos_v.at[pl.ds(k * sub, sub)], psem))

        stores = []
        for k in range(_NSUB):
            gathers[k].wait()
            poscps[k].wait()

            def add_rows(i, _, k=k):
                r = k * sub + i
                for j in range(dim // _LANES):
                    sl = pl.ds(j * _LANES, _LANES)
                    plsc.addupdate(rows_v.at[r, sl], pos_v[r, sl])
                return 0

            lax.fori_loop(0, sub, add_rows, 0)
            stores.append(pltpu.async_copy(
                rows_v.at[pl.ds(k * sub, sub)],
                out_hbm.at[b, pl.ds(col + k * sub, sub)], osem))
        for st in stores:
            st.wait()

    return embed


def kernel(tok_idx, tok_table, pos_table):
    bs, seq_len = tok_idx.shape
    dim = tok_table.shape[1]
    embed = _build(bs, seq_len, dim)
    return embed(tok_idx.astype(jnp.int32), tok_table, pos_table)


# speedup vs baseline: 1.2628x; 1.0049x over previous
import functools

import jax
import jax.numpy as jnp
from jax import lax
from jax.experimental import pallas as pl
from jax.experimental.pallas import tpu as pltpu
from jax.experimental.pallas import tpu_sc as plsc

_LANES = 16
_ROW_UNROLL = 1
_NSUB = 4


@functools.lru_cache(maxsize=None)
def _build(bs: int, seq_len: int, dim: int):
    info = plsc.get_sparse_core_info()
    nc, ns = info.num_cores, info.num_subcores
    nw = nc * ns
    num_rows = bs * seq_len
    assert num_rows % (nw * _NSUB) == 0
    chunk = num_rows // nw
    sub = chunk // _NSUB
    assert sub % 8 == 0 and sub % _ROW_UNROLL == 0
    assert seq_len % chunk == 0 and dim % _LANES == 0
    tiles_per_b = seq_len // chunk
    nslots = seq_len // (chunk * nc)
    assert nslots >= 1 and ns % nslots == 0

    mesh = plsc.VectorSubcoreMesh(core_axis_name="c", subcore_axis_name="s")

    @functools.partial(
        pl.kernel,
        mesh=mesh,
        out_type=jax.ShapeDtypeStruct((bs, seq_len, dim), jnp.float32),
        scratch_types=[
            pltpu.VMEM((chunk,), jnp.int32),
            pltpu.VMEM((chunk, dim), jnp.float32),
            pltpu.VMEM((chunk, dim), jnp.float32),
            pltpu.VMEM_SHARED((nslots, chunk, dim), jnp.float32),
            pltpu.SemaphoreType.DMA,
            pltpu.SemaphoreType.DMA,
            pltpu.SemaphoreType.DMA,
            pltpu.SemaphoreType.DMA,
        ],
    )
    def embed(idx_hbm, tok_hbm, pos_hbm, out_hbm, idx_v, rows_v, pos_v,
              pos_sh, gsem, osem, ssem, psem):
        cid = lax.axis_index("c")
        sid = lax.axis_index("s")
        wid = sid * nc + cid
        b = wid // tiles_per_b
        col = (wid % tiles_per_b) * chunk

        pltpu.sync_copy(idx_hbm.at[b, pl.ds(col, chunk)], idx_v)
        gathers = []
        for k in range(_NSUB):
            gathers.append(pltpu.async_copy(
                tok_hbm.at[idx_v.at[pl.ds(k * sub, sub)]],
                rows_v.at[pl.ds(k * sub, sub)], gsem))

        @pl.when(sid < nslots)
        def _stage():
            start = (nc * sid + cid) * chunk
            pltpu.async_copy(
                pos_hbm.at[pl.ds(start, chunk)], pos_sh.at[sid], ssem).wait()

        plsc.subcore_barrier()
        poscps = []
        for k in range(_NSUB):
            poscps.append(pltpu.async_copy(
                pos_sh.at[sid % nslots, pl.ds(k * sub, sub)],
                pos_v.at[pl.ds(k * sub, sub)], psem))

        stores = []
        for k in range(_NSUB):
            gathers[k].wait()
            poscps[k].wait()

            def add_rows(i, _, k=k):
                r = k * sub + i
                for j in range(dim // _LANES):
                    sl = pl.ds(j * _LANES, _LANES)
                    plsc.addupdate(rows_v.at[r, sl], pos_v[r, sl])
                return 0

            lax.fori_loop(0, sub, add_rows, 0)
            stores.append(pltpu.async_copy(
                rows_v.at[pl.ds(k * sub, sub)],
                out_hbm.at[b, pl.ds(col + k * sub, sub)], osem))
        for st in stores:
            st.wait()

    return embed


def kernel(tok_idx, tok_table, pos_table):
    bs, seq_len = tok_idx.shape
    dim = tok_table.shape[1]
    embed = _build(bs, seq_len, dim)
    return embed(tok_idx.astype(jnp.int32), tok_table, pos_table)
